# baseline (speedup 1.0000x reference)
import functools

import jax
import jax.numpy as jnp
from jax import lax
from jax.experimental import pallas as pl
from jax.experimental.pallas import tpu as pltpu
from jax.experimental.pallas import tpu_sc as plsc

_B = 16384
_D = 64
_O = 128
_NC = 2
_NS = 16
_NW = _NC * _NS
_BPW = _B // _NW


def _sc_gather(table, idx):
  mesh = plsc.VectorSubcoreMesh(core_axis_name="c", subcore_axis_name="s")

  @functools.partial(
      pl.kernel,
      mesh=mesh,
      out_type=[
          jax.ShapeDtypeStruct((_B, _D), jnp.float32),
          jax.ShapeDtypeStruct((_B, _D), jnp.float32),
      ],
      scratch_types=[
          pltpu.VMEM((2 * _BPW,), jnp.int32),
          pltpu.SemaphoreType.DMA,
      ],
      compiler_params=pltpu.CompilerParams(use_tc_tiling_on_sc=True),
  )
  def gather_kernel(table_hbm, idx_hbm, e1_hbm, e2_hbm, idx_v, sem):
    wid = lax.axis_index("s") * _NC + lax.axis_index("c")
    base = wid * _BPW
    pltpu.sync_copy(idx_hbm.at[pl.ds(2 * base, 2 * _BPW)], idx_v)

    def body(g, carry):
      v = idx_v[pl.ds(16 * g, 16)]
      k = base + 8 * g
      for j in range(16):
        dst = e1_hbm if j % 2 == 0 else e2_hbm
        pltpu.async_copy(table_hbm.at[pl.ds(v[j], 1)],
                         dst.at[pl.ds(k + j // 2, 1)], sem)
      return carry

    lax.fori_loop(0, 2 * _BPW // 16, body, 0)
    pltpu.make_async_copy(table_hbm.at[pl.ds(0, _BPW)],
                          e1_hbm.at[pl.ds(base, _BPW)], sem).wait()
    pltpu.make_async_copy(table_hbm.at[pl.ds(0, _BPW)],
                          e2_hbm.at[pl.ds(base, _BPW)], sem).wait()

  return gather_kernel(table, idx)


_BM = 1024


def _tc_project(e1, e2, W, b2d):
  def mm_kernel(e1_ref, e2_ref, w_ref, b_ref, o_ref):
    acc = lax.dot_general(e1_ref[...], w_ref[:, :_D],
                          (((1,), (1,)), ((), ())),
                          preferred_element_type=jnp.float32)
    acc += lax.dot_general(e2_ref[...], w_ref[:, _D:],
                           (((1,), (1,)), ((), ())),
                           preferred_element_type=jnp.float32)
    o_ref[...] = acc + b_ref[...]

  return pl.pallas_call(
      mm_kernel,
      grid=(_B // _BM,),
      in_specs=[
          pl.BlockSpec((_BM, _D), lambda i: (i, 0)),
          pl.BlockSpec((_BM, _D), lambda i: (i, 0)),
          pl.BlockSpec((_O, 2 * _D), lambda i: (0, 0)),
          pl.BlockSpec((1, _O), lambda i: (0, 0)),
      ],
      out_specs=pl.BlockSpec((_BM, _O), lambda i: (i, 0)),
      out_shape=jax.ShapeDtypeStruct((_B, _O), jnp.float32),
  )(e1, e2, W, b2d)


def kernel(x, emb_table, W, b):
  idx = x.astype(jnp.int32).reshape(2 * _B)
  e1, e2 = _sc_gather(emb_table, idx)
  return _tc_project(e1, e2, W, b.reshape(1, _O))


# speedup vs baseline: 5.7299x; 5.7299x over previous
import functools

import jax
import jax.numpy as jnp
from jax import lax
from jax.experimental import pallas as pl
from jax.experimental.pallas import tpu as pltpu
from jax.experimental.pallas import tpu_sc as plsc

_B = 16384
_D = 64
_O = 128
_G = 2 * _B
_NC = 2
_NS = 16
_NW = _NC * _NS
_BPW = _B // _NW
_RPW = 2 * _BPW
_CH = 128
_NCH = _RPW // _CH


def _sc_gather(table, idx1, idx2):
  mesh = plsc.VectorSubcoreMesh(core_axis_name="c", subcore_axis_name="s")

  @functools.partial(
      pl.kernel,
      mesh=mesh,
      out_type=jax.ShapeDtypeStruct((_G, _D), jnp.float32),
      scratch_types=[
          pltpu.VMEM((_BPW,), jnp.int32),
          pltpu.VMEM((_BPW,), jnp.int32),
          pltpu.VMEM((_NCH, _CH), jnp.int32),
          pltpu.VMEM((_RPW, _D), jnp.float32),
          pltpu.SemaphoreType.DMA,
      ],
      compiler_params=pltpu.CompilerParams(use_tc_tiling_on_sc=False,
                                           needs_layout_passes=False),
  )
  def gather_kernel(table_hbm, idx1_hbm, idx2_hbm, out_hbm,
                    i1_v, i2_v, cat_v, rows_v, sem):
    wid = lax.axis_index("s") * _NC + lax.axis_index("c")
    base = wid * _BPW
    pltpu.sync_copy(idx1_hbm.at[pl.ds(base, _BPW)], i1_v)
    pltpu.sync_copy(idx2_hbm.at[pl.ds(base, _BPW)], i2_v)
    lanes2 = 2 * lax.iota(jnp.int32, 16)
    for k in range(_BPW // 16):
      v1 = i1_v[pl.ds(16 * k, 16)]
      v2 = i2_v[pl.ds(16 * k, 16)]
      row = (32 * k) // _CH
      col0 = (32 * k) % _CH
      plsc.store_scatter(cat_v.at[row], [col0 + lanes2], v1)
      plsc.store_scatter(cat_v.at[row], [col0 + 1 + lanes2], v2)
    copies = []
    for j in range(_NCH):
      copies.append(pltpu.async_copy(
          table_hbm.at[cat_v.at[j]], rows_v.at[pl.ds(j * _CH, _CH)], sem))
    for c in copies:
      c.wait()
    pltpu.sync_copy(rows_v, out_hbm.at[pl.ds(wid * _RPW, _RPW)])

  return gather_kernel(table, idx1, idx2)


_BM = 1024


def _tc_project(cat, W, b2d):
  def mm_kernel(cat_ref, w_ref, b_ref, o_ref):
    o_ref[...] = lax.dot_general(
        cat_ref[...], w_ref[...], (((1,), (1,)), ((), ())),
        preferred_element_type=jnp.float32) + b_ref[...]

  return pl.pallas_call(
      mm_kernel,
      grid=(_B // _BM,),
      in_specs=[
          pl.BlockSpec((_BM, 2 * _D), lambda i: (i, 0)),
          pl.BlockSpec((_O, 2 * _D), lambda i: (0, 0)),
          pl.BlockSpec((1, _O), lambda i: (0, 0)),
      ],
      out_specs=pl.BlockSpec((_BM, _O), lambda i: (i, 0)),
      out_shape=jax.ShapeDtypeStruct((_B, _O), jnp.float32),
  )(cat, W, b2d)


def kernel(x, emb_table, W, b):
  xT = x.T.astype(jnp.int32)
  rows = _sc_gather(emb_table, xT[0], xT[1])
  cat = rows.reshape(_B, 2 * _D)
  return _tc_project(cat, W, b.reshape(1, _O))
